# baseline (speedup 1.0000x reference)
import functools

import jax
import jax.numpy as jnp
from jax import lax
from jax.experimental import pallas as pl
from jax.experimental.pallas import tpu as pltpu
from jax.experimental.pallas import tpu_sc as plsc

_K = 128
_L = 16
_JL = 8


def _lane_shuffle(x, idx_lanes):
    dnums = lax.GatherDimensionNumbers(
        offset_dims=(), collapsed_slice_dims=(0,), start_index_map=(0,))
    return lax.gather(x, idx_lanes[:, None], dnums, (1,),
                      mode=lax.GatherScatterMode.PROMISE_IN_BOUNDS)


def _glove_body(i_hbm, j_hbm, table_hbm, bvec_hbm, out_hbm,
                idx_v, rows_v, bv_v, out_v, sem_w, sem_b):
    @pl.when(jnp.logical_and(lax.axis_index("c") == 0,
                             lax.axis_index("s") == 0))
    def _():
        idx_v[...] = jnp.zeros((_L,), jnp.int32)
        pltpu.sync_copy(i_hbm, idx_v.at[pl.ds(0, 1)])
        pltpu.sync_copy(j_hbm, idx_v.at[pl.ds(_JL, 1)])
        cp_w = pltpu.async_copy(table_hbm.at[idx_v], rows_v, sem_w)
        cp_b = pltpu.async_copy(bvec_hbm.at[idx_v], bv_v, sem_b)
        cp_w.wait()
        cp_b.wait()
        lane = lax.iota(jnp.int32, _L)
        sel = jnp.logical_or(lane == 0, lane == _JL)
        acc = jnp.where(sel, bv_v[...], 0.0)
        for c in range(_K // _L):
            acc = acc + (rows_v[0, pl.ds(c * _L, _L)]
                         * rows_v[_JL, pl.ds(c * _L, _L)])
        for sh in (8, 4, 2, 1):
            acc = acc + _lane_shuffle(acc, (lane + sh) & (_L - 1))
        out_v[...] = acc
        pltpu.sync_copy(out_v, out_hbm)


_glove_sc = functools.partial(
    pl.kernel,
    out_type=jax.ShapeDtypeStruct((_L,), jnp.float32),
    mesh=plsc.VectorSubcoreMesh(core_axis_name="c", subcore_axis_name="s",
                                num_cores=1, num_subcores=1),
    scratch_types=[
        pltpu.VMEM((_L,), jnp.int32),
        pltpu.VMEM((_L, _K), jnp.float32),
        pltpu.VMEM((_L,), jnp.float32),
        pltpu.VMEM((_L,), jnp.float32),
        pltpu.SemaphoreType.DMA,
        pltpu.SemaphoreType.DMA,
    ],
)(_glove_body)


def kernel(W, B, i, j):
    i1 = jnp.reshape(jnp.asarray(i, jnp.int32), (1,))
    j1 = jnp.reshape(jnp.asarray(j, jnp.int32), (1,))
    out = _glove_sc(i1, j1, W, B)
    return out[0]


# speedup vs baseline: 1.0709x; 1.0709x over previous
import functools

import jax
import jax.numpy as jnp
from jax import lax
from jax.experimental import pallas as pl
from jax.experimental.pallas import tpu as pltpu
from jax.experimental.pallas import tpu_sc as plsc

_K = 128
_L = 16
_JL = 8


def _lane_shuffle(x, idx_lanes):
    dnums = lax.GatherDimensionNumbers(
        offset_dims=(), collapsed_slice_dims=(0,), start_index_map=(0,))
    return lax.gather(x, idx_lanes[:, None], dnums, (1,),
                      mode=lax.GatherScatterMode.PROMISE_IN_BOUNDS)


def _glove_body(i_hbm, j_hbm, table_hbm, bvec_hbm, out_hbm,
                stage_v, idx_v, rows_v, bv_v, out_v, sem_i, sem_j):
    @pl.when(jnp.logical_and(lax.axis_index("c") == 0,
                             lax.axis_index("s") == 0))
    def _():
        lane = lax.iota(jnp.int32, _L)
        cp_i = pltpu.async_copy(i_hbm, stage_v.at[pl.ds(0, 1)], sem_i)
        cp_j = pltpu.async_copy(j_hbm, stage_v.at[pl.ds(_JL, 1)], sem_j)
        cp_i.wait()
        cp_j.wait()
        idx_v[...] = _lane_shuffle(stage_v[...],
                                   jnp.where(lane == 0, 0, _JL))
        cp_w = pltpu.async_copy(table_hbm.at[idx_v.at[pl.ds(0, 2)]],
                                rows_v, sem_i)
        cp_b = pltpu.async_copy(bvec_hbm.at[idx_v], bv_v, sem_j)
        cp_w.wait()
        cp_b.wait()
        acc = jnp.where(lane < 2, bv_v[...], 0.0)
        for c in range(_K // _L):
            acc = acc + (rows_v[0, pl.ds(c * _L, _L)]
                         * rows_v[1, pl.ds(c * _L, _L)])
        for sh in (8, 4, 2, 1):
            acc = acc + _lane_shuffle(acc, (lane + sh) & (_L - 1))
        out_v[...] = acc
        pltpu.sync_copy(out_v, out_hbm)


_glove_sc = functools.partial(
    pl.kernel,
    out_type=jax.ShapeDtypeStruct((_L,), jnp.float32),
    mesh=plsc.VectorSubcoreMesh(core_axis_name="c", subcore_axis_name="s",
                                num_cores=1, num_subcores=1),
    scratch_types=[
        pltpu.VMEM((_L,), jnp.int32),
        pltpu.VMEM((_L,), jnp.int32),
        pltpu.VMEM((2, _K), jnp.float32),
        pltpu.VMEM((_L,), jnp.float32),
        pltpu.VMEM((_L,), jnp.float32),
        pltpu.SemaphoreType.DMA,
        pltpu.SemaphoreType.DMA,
    ],
)(_glove_body)


def kernel(W, B, i, j):
    i1 = jnp.reshape(jnp.asarray(i, jnp.int32), (1,))
    j1 = jnp.reshape(jnp.asarray(j, jnp.int32), (1,))
    out = _glove_sc(i1, j1, W, B)
    return out[0]
